# baseline (speedup 1.0000x reference)
import functools

import jax
import jax.numpy as jnp
from jax.experimental import pallas as pl
from jax.experimental.pallas import tpu as pltpu

H = 4096
H4 = H // 4
BT = 512


def _body(x_ref, w_ref, b1_ref, wd_ref, c_ref, o_ref):
    xb = x_ref[...].astype(jnp.bfloat16)
    h = jax.lax.dot_general(
        xb, w_ref[...], (((1,), (0,)), ((), ())),
        preferred_element_type=jnp.float32,
    )
    h = h + b1_ref[...]
    h = h * jax.nn.sigmoid(h)
    t = jnp.sum(h * wd_ref[...], axis=1) + c_ref[0]
    o_ref[...] = jax.nn.sigmoid(t)[None, None, :]


@functools.partial(jax.jit, static_argnames=())
def kernel(hidden_states, layer_idx, W1, b1, W2, b2, layer_bias):
    orig_shape = hidden_states.shape[:-1]
    x = hidden_states.reshape(-1, H)
    n = x.shape[0]
    nb = n // BT

    w1t = W1.T.astype(jnp.bfloat16)
    wd = (W2[1] - W2[0]).reshape(1, H4)
    c = (b2[1] - b2[0] + layer_bias[layer_idx]).reshape(1)
    b1r = b1.reshape(1, H4)

    out = pl.pallas_call(
        _body,
        grid=(nb,),
        in_specs=[
            pl.BlockSpec((BT, H), lambda i: (i, 0)),
            pl.BlockSpec((H, H4), lambda i: (0, 0)),
            pl.BlockSpec((1, H4), lambda i: (0, 0)),
            pl.BlockSpec((1, H4), lambda i: (0, 0)),
            pl.BlockSpec(memory_space=pltpu.SMEM),
        ],
        out_specs=pl.BlockSpec((1, 1, BT), lambda i: (i, 0, 0)),
        out_shape=jax.ShapeDtypeStruct((nb, 1, BT), jnp.float32),
        compiler_params=pltpu.CompilerParams(
            dimension_semantics=("parallel",),
        ),
    )(x, w1t, b1r, wd, c)
    return out.reshape(orig_shape)


# speedup vs baseline: 1.0277x; 1.0277x over previous
import functools

import jax
import jax.numpy as jnp
from jax.experimental import pallas as pl
from jax.experimental.pallas import tpu as pltpu

H = 4096
H4 = H // 4
BT = 512


NCHUNK = 4


def _body(x_ref, w_ref, b1_ref, wd_ref, c_ref, o_ref):
    w = w_ref[...]
    mc = BT // NCHUNK
    for j in range(NCHUNK):
        xb = x_ref[pl.ds(j * mc, mc), :].astype(jnp.bfloat16)
        h = jax.lax.dot_general(
            xb, w, (((1,), (0,)), ((), ())),
            preferred_element_type=jnp.float32,
        )
        h = h + b1_ref[...]
        h = h * jax.nn.sigmoid(h)
        t = jnp.sum(h * wd_ref[...], axis=1) + c_ref[0]
        o_ref[0, 0, pl.ds(j * mc, mc)] = jax.nn.sigmoid(t)


@functools.partial(jax.jit, static_argnames=())
def kernel(hidden_states, layer_idx, W1, b1, W2, b2, layer_bias):
    orig_shape = hidden_states.shape[:-1]
    x = hidden_states.reshape(-1, H)
    n = x.shape[0]
    nb = n // BT

    w1t = W1.T.astype(jnp.bfloat16)
    wd = (W2[1] - W2[0]).reshape(1, H4)
    c = (b2[1] - b2[0] + layer_bias[layer_idx]).reshape(1)
    b1r = b1.reshape(1, H4)

    out = pl.pallas_call(
        _body,
        grid=(nb,),
        in_specs=[
            pl.BlockSpec((BT, H), lambda i: (i, 0)),
            pl.BlockSpec((H, H4), lambda i: (0, 0)),
            pl.BlockSpec((1, H4), lambda i: (0, 0)),
            pl.BlockSpec((1, H4), lambda i: (0, 0)),
            pl.BlockSpec(memory_space=pltpu.SMEM),
        ],
        out_specs=pl.BlockSpec((1, 1, BT), lambda i: (i, 0, 0)),
        out_shape=jax.ShapeDtypeStruct((nb, 1, BT), jnp.float32),
        compiler_params=pltpu.CompilerParams(
            dimension_semantics=("parallel",),
        ),
    )(x, w1t, b1r, wd, c)
    return out.reshape(orig_shape)


# speedup vs baseline: 1.0832x; 1.0539x over previous
import functools

import jax
import jax.numpy as jnp
from jax.experimental import pallas as pl
from jax.experimental.pallas import tpu as pltpu

H = 4096
H4 = H // 4
BT = 1024


NCHUNK = 8


def _body(x_ref, w_ref, b1_ref, wd_ref, c_ref, o_ref):
    w = w_ref[...]
    mc = BT // NCHUNK
    for j in range(NCHUNK):
        xb = x_ref[pl.ds(j * mc, mc), :].astype(jnp.bfloat16)
        h = jax.lax.dot_general(
            xb, w, (((1,), (0,)), ((), ())),
            preferred_element_type=jnp.float32,
        )
        h = h + b1_ref[...]
        h = h * jax.nn.sigmoid(h)
        t = jnp.sum(h * wd_ref[...], axis=1) + c_ref[0]
        o_ref[0, 0, pl.ds(j * mc, mc)] = jax.nn.sigmoid(t)


@functools.partial(jax.jit, static_argnames=())
def kernel(hidden_states, layer_idx, W1, b1, W2, b2, layer_bias):
    orig_shape = hidden_states.shape[:-1]
    x = hidden_states.reshape(-1, H)
    n = x.shape[0]
    nb = n // BT

    w1t = W1.T.astype(jnp.bfloat16)
    wd = (W2[1] - W2[0]).reshape(1, H4)
    c = (b2[1] - b2[0] + layer_bias[layer_idx]).reshape(1)
    b1r = b1.reshape(1, H4)

    out = pl.pallas_call(
        _body,
        grid=(nb,),
        in_specs=[
            pl.BlockSpec((BT, H), lambda i: (i, 0)),
            pl.BlockSpec((H, H4), lambda i: (0, 0)),
            pl.BlockSpec((1, H4), lambda i: (0, 0)),
            pl.BlockSpec((1, H4), lambda i: (0, 0)),
            pl.BlockSpec(memory_space=pltpu.SMEM),
        ],
        out_specs=pl.BlockSpec((1, 1, BT), lambda i: (i, 0, 0)),
        out_shape=jax.ShapeDtypeStruct((nb, 1, BT), jnp.float32),
        compiler_params=pltpu.CompilerParams(
            dimension_semantics=("parallel",),
        ),
    )(x, w1t, b1r, wd, c)
    return out.reshape(orig_shape)
